# baseline (speedup 1.0000x reference)
import jax
import jax.numpy as jnp
from jax.experimental import pallas as pl
from jax.experimental.pallas import tpu as pltpu

N = 4096
IN_C = 128
HID1 = 64
HID2 = 32
BLK = 512
NB = N // BLK

def _body(x_ref, adj_ref, W1_ref, Wmu_ref, Q_out, P_ref):
    i = pl.program_id(0)

    @pl.when(i == 0)
    def _init():
        P_ref[...] = jnp.dot(x_ref[...], W1_ref[...],
                             preferred_element_type=jnp.float32)

    h = jax.nn.relu(jax.lax.dot_general(
        adj_ref[...], P_ref[...],
        dimension_numbers=(((1,), (0,)), ((), ())),
        precision=jax.lax.Precision.DEFAULT,
        preferred_element_type=jnp.float32))
    Q_out[...] = jnp.dot(h, Wmu_ref[...],
                         preferred_element_type=jnp.float32).astype(jnp.bfloat16)

def kernel(x, adj, W1, W_mu, W_var):
    return pl.pallas_call(
        _body,
        grid=(NB,),
        in_specs=[
            pl.BlockSpec((N, IN_C), lambda i: (0, 0)),
            pl.BlockSpec((BLK, N), lambda i: (i, 0)),
            pl.BlockSpec((IN_C, HID1), lambda i: (0, 0)),
            pl.BlockSpec((HID1, HID2), lambda i: (0, 0)),
        ],
        out_specs=pl.BlockSpec((BLK, HID2), lambda i: (i, 0)),
        out_shape=jax.ShapeDtypeStruct((N, HID2), jnp.bfloat16),
        scratch_shapes=[pltpu.VMEM((N, HID1), jnp.float32)],
    )(x, adj, W1, W_mu)
